# Initial kernel scaffold; baseline (speedup 1.0000x reference)
import jax
import jax.numpy as jnp
from jax.experimental import pallas as pl


def kernel(node_feats, edge_feats, rigids, node_mask, Wn0, bn0, Wn1, bn1, Wn2, bn2, We0, be0, We1, be1, We2, be2, Wma, Wmb, Wmz, Wmr, bm, Wm1, bm1, Wnu, bnu, Wea, Web, Wez, beu, Ws, edge_index, batch):
    raise NotImplementedError("write your pallas kernel here")


# speedup vs baseline: 2.7348x; 2.7348x over previous
import functools

import jax
import jax.numpy as jnp
from jax import lax
from jax.experimental import pallas as pl
from jax.experimental.pallas import tpu as pltpu
from jax.experimental.pallas import tpu_sc as plsc

N = 10000
NP = 10240
E = 160000
C_IN = 69
C_S = 256
C_Z = 128
C_H = 256
L = 4
B = 8

NBLK = 256
EBLK = 1000
F32 = jnp.float32


def _ln(x, eps=1e-5):
    m = jnp.mean(x, axis=-1, keepdims=True)
    xc = x - m
    v = jnp.mean(xc * xc, axis=-1, keepdims=True)
    return xc / jnp.sqrt(v + eps)


def _full(shape):
    return pl.BlockSpec(shape, lambda i: (0,) * len(shape))


def _rows(shape):
    return pl.BlockSpec(shape, lambda i: (i,) + (0,) * (len(shape) - 1))


def _node_embed_body(nf, w0, b0, w1, b1, w2, b2, s_ref):
    h = jnp.maximum(jnp.dot(nf[...], w0[...], preferred_element_type=F32) + b0[...], 0.0)
    h = jnp.maximum(jnp.dot(h, w1[...], preferred_element_type=F32) + b1[...], 0.0)
    s_ref[...] = _ln(jnp.dot(h, w2[...], preferred_element_type=F32) + b2[...])


def _node_embed(nf, w0, b0, w1, b1, w2, b2):
    return pl.pallas_call(
        _node_embed_body,
        grid=(NP // NBLK,),
        in_specs=[_rows((NBLK, C_IN)), _full((C_IN, 2 * C_S)), _full((1, 2 * C_S)),
                  _full((2 * C_S, 2 * C_S)), _full((1, 2 * C_S)),
                  _full((2 * C_S, C_S)), _full((1, C_S))],
        out_specs=_rows((NBLK, C_S)),
        out_shape=jax.ShapeDtypeStruct((NP, C_S), F32),
    )(nf, w0, b0, w1, b1, w2, b2)


def _edge_embed_body(ef, rs, rd, w0, b0, w1, b1, w2, b2, z_ref, rel_ref):
    g = jnp.maximum(jnp.dot(ef[...], w0[...], preferred_element_type=F32) + b0[...], 0.0)
    g = jnp.maximum(jnp.dot(g, w1[...], preferred_element_type=F32) + b1[...], 0.0)
    z_ref[...] = _ln(jnp.dot(g, w2[...], preferred_element_type=F32) + b2[...])
    rdv = rd[...]
    rsv = rs[...]
    qw = rdv[:, 0:1]; qx = rdv[:, 1:2]; qy = rdv[:, 2:3]; qz = rdv[:, 3:4]
    r = 1.0 / (jnp.sqrt(qw * qw + qx * qx + qy * qy + qz * qz) + 1e-8)
    w = qw * r; ux = -qx * r; uy = -qy * r; uz = -qz * r
    vx = rsv[:, 4:5] - rdv[:, 4:5]
    vy = rsv[:, 5:6] - rdv[:, 5:6]
    vz = rsv[:, 6:7] - rdv[:, 6:7]
    uvx = uy * vz - uz * vy
    uvy = uz * vx - ux * vz
    uvz = ux * vy - uy * vx
    cx = uy * uvz - uz * uvy
    cy = uz * uvx - ux * uvz
    cz = ux * uvy - uy * uvx
    relx = vx + 2.0 * (w * uvx + cx)
    rely = vy + 2.0 * (w * uvy + cy)
    relz = vz + 2.0 * (w * uvz + cz)
    rel_ref[...] = jnp.concatenate([relx, rely, relz, jnp.zeros_like(relx)], axis=1)


def _edge_embed(ef, rs, rd, w0, b0, w1, b1, w2, b2):
    return pl.pallas_call(
        _edge_embed_body,
        grid=(E // EBLK,),
        in_specs=[_rows((EBLK, 128)), _rows((EBLK, 128)), _rows((EBLK, 128)),
                  _full((128, 2 * C_Z)), _full((1, 2 * C_Z)),
                  _full((2 * C_Z, 2 * C_Z)), _full((1, 2 * C_Z)),
                  _full((2 * C_Z, C_Z)), _full((1, C_Z))],
        out_specs=[_rows((EBLK, C_Z)), _rows((EBLK, 4))],
        out_shape=[jax.ShapeDtypeStruct((E, C_Z), F32),
                   jax.ShapeDtypeStruct((E, 4), F32)],
    )(ef, rs, rd, w0, b0, w1, b1, w2, b2)


def _ab_body(s, wma, wmb, a_ref, b_ref):
    a_ref[...] = jnp.dot(s[...], wma[...], preferred_element_type=F32)
    b_ref[...] = jnp.dot(s[...], wmb[...], preferred_element_type=F32)


def _ab(s, wma, wmb):
    return pl.pallas_call(
        _ab_body,
        grid=(NP // NBLK,),
        in_specs=[_rows((NBLK, C_S)), _full((C_S, C_H)), _full((C_S, C_H))],
        out_specs=[_rows((NBLK, C_H)), _rows((NBLK, C_H))],
        out_shape=[jax.ShapeDtypeStruct((NP, C_H), F32),
                   jax.ShapeDtypeStruct((NP, C_H), F32)],
    )(s, wma, wmb)


def _msg_body(asrc, bdst, z, rel, wmz, wr, bm, wm1, bm1, mlo_ref, mhi_ref):
    pre = asrc[...] + bdst[...] + bm[...]
    pre += jnp.dot(z[...], wmz[...], preferred_element_type=F32)
    pre += jnp.dot(rel[...], wr[...], preferred_element_type=F32)
    m = jnp.dot(jnp.maximum(pre, 0.0), wm1[...], preferred_element_type=F32) + bm1[...]
    mlo_ref[...] = m[:, :128]
    mhi_ref[...] = m[:, 128:]


def _msg(asrc, bdst, z, rel, wmz, wr, bm, wm1, bm1):
    return pl.pallas_call(
        _msg_body,
        grid=(E // EBLK,),
        in_specs=[_rows((EBLK, C_H)), _rows((EBLK, C_H)), _rows((EBLK, C_Z)),
                  _rows((EBLK, 4)), _full((C_Z, C_H)), _full((4, C_H)),
                  _full((1, C_H)), _full((C_H, C_S)), _full((1, C_S))],
        out_specs=[_rows((EBLK, 128)), _rows((EBLK, 128))],
        out_shape=[jax.ShapeDtypeStruct((E, 128), F32),
                   jax.ShapeDtypeStruct((E, 128), F32)],
    )(asrc, bdst, z, rel, wmz, wr, bm, wm1, bm1)


def _node_upd_body(s, p0l, p1l, p0h, p1h, wnu, bnu, wea, web, mask,
                   s_ref, ea_ref, eb_ref):
    agg = jnp.concatenate([p0l[...] + p1l[...], p0h[...] + p1h[...]], axis=1)
    sn = _ln(s[...] + jnp.dot(agg, wnu[...], preferred_element_type=F32) + bnu[...])
    sn = sn * mask[...]
    s_ref[...] = sn
    ea_ref[...] = jnp.dot(sn, wea[...], preferred_element_type=F32)
    eb_ref[...] = jnp.dot(sn, web[...], preferred_element_type=F32)


def _node_upd(s, p0l, p1l, p0h, p1h, wnu, bnu, wea, web, mask):
    return pl.pallas_call(
        _node_upd_body,
        grid=(NP // NBLK,),
        in_specs=[_rows((NBLK, C_S)),
                  _rows((NBLK, 128)), _rows((NBLK, 128)),
                  _rows((NBLK, 128)), _rows((NBLK, 128)),
                  _full((C_S, C_S)), _full((1, C_S)),
                  _full((C_S, C_Z)), _full((C_S, C_Z)), _rows((NBLK, 1))],
        out_specs=[_rows((NBLK, C_S)), _rows((NBLK, C_Z)), _rows((NBLK, C_Z))],
        out_shape=[jax.ShapeDtypeStruct((NP, C_S), F32),
                   jax.ShapeDtypeStruct((NP, C_Z), F32),
                   jax.ShapeDtypeStruct((NP, C_Z), F32)],
    )(s, p0l, p1l, p0h, p1h, wnu, bnu, wea, web, mask)


def _z_upd_body(z, eas, ebd, wez, beu, z_ref):
    zv = z[...]
    z_ref[...] = _ln(zv + eas[...] + ebd[...] + beu[...]
                     + jnp.dot(zv, wez[...], preferred_element_type=F32))


def _z_upd(z, eas, ebd, wez, beu):
    return pl.pallas_call(
        _z_upd_body,
        grid=(E // EBLK,),
        in_specs=[_rows((EBLK, C_Z)), _rows((EBLK, C_Z)), _rows((EBLK, C_Z)),
                  _full((C_Z, C_Z)), _full((1, C_Z))],
        out_specs=_rows((EBLK, C_Z)),
        out_shape=jax.ShapeDtypeStruct((E, C_Z), F32),
    )(z, eas, ebd, wez, beu)


def _pool_body(s, mask, batch, ws, pooled_ref, denom_ref, score_ref):
    i = pl.program_id(0)

    @pl.when(i == 0)
    def _():
        pooled_ref[...] = jnp.zeros_like(pooled_ref)
        denom_ref[...] = jnp.zeros_like(denom_ref)

    b = batch[0]
    onehot = (b == lax.broadcasted_iota(jnp.int32, (B, NBLK), 0)).astype(F32)
    mv = mask[...]
    pooled_ref[...] += jnp.dot(onehot, s[...] * mv, preferred_element_type=F32)
    denom_ref[...] += jnp.dot(onehot, mv, preferred_element_type=F32)

    @pl.when(i == pl.num_programs(0) - 1)
    def _():
        denom = jnp.clip(denom_ref[...], 1.0, None)
        score_ref[...] = jnp.dot(_ln(pooled_ref[...] / denom), ws[...],
                                 preferred_element_type=F32)


def _pool(s, mask, batch3, ws):
    return pl.pallas_call(
        _pool_body,
        grid=(NP // NBLK,),
        in_specs=[_rows((NBLK, C_S)), _rows((NBLK, 1)),
                  pl.BlockSpec((1, 1, NBLK), lambda i: (i, 0, 0)),
                  _full((C_S, 1))],
        out_specs=[_full((B, C_S)), _full((B, 1)), _full((B, 1))],
        out_shape=[jax.ShapeDtypeStruct((B, C_S), F32),
                   jax.ShapeDtypeStruct((B, 1), F32),
                   jax.ShapeDtypeStruct((B, 1), F32)],
    )(s, mask, batch3, ws)


_GW = 128


def _sc_gather(table, idx):
    d = table.shape[1]
    n_idx = idx.shape[0]
    mesh = plsc.VectorSubcoreMesh(core_axis_name="c", subcore_axis_name="s")

    @functools.partial(
        pl.kernel,
        out_type=jax.ShapeDtypeStruct((n_idx, d), F32),
        mesh=mesh,
    )
    def k(tab_hbm, i_hbm, o_hbm):
        def body(i_vmem, o_vmem):
            pltpu.sync_copy(tab_hbm.at[i_vmem.at[0]], o_vmem)

        pltpu.emit_pipeline(
            body,
            grid=(n_idx // _GW,),
            in_specs=[pl.BlockSpec((1, _GW), lambda i: (0, i))],
            out_specs=[pl.BlockSpec((_GW, d), lambda i: (i, 0))],
            core_axis_name=("c", "s"),
            dimension_semantics=(pltpu.PARALLEL,),
        )(i_hbm, o_hbm)

    return k(table, idx.reshape(1, n_idx))


_SCH = 200


def _sc_scatter_add(m, idx, zrows):
    d = m.shape[1]
    per_w = E // 32
    n_ch = per_w // _SCH
    rows_per_sub = NP // 16
    mesh = plsc.VectorSubcoreMesh(core_axis_name="c", subcore_axis_name="s")

    @functools.partial(
        pl.kernel,
        out_type=jax.ShapeDtypeStruct((2, NP, d), F32),
        mesh=mesh,
        scratch_types=[pltpu.VMEM((_SCH,), jnp.int32),
                       pltpu.VMEM((_SCH, d), F32),
                       pltpu.VMEM_SHARED((NP, d), F32)],
    )
    def k(m_hbm, i_hbm, z_hbm, o_hbm, idx_v, rows_v, acc):
        cid = lax.axis_index("c")
        sid = lax.axis_index("s")
        pltpu.sync_copy(z_hbm, acc.at[pl.ds(sid * rows_per_sub, rows_per_sub)])
        plsc.subcore_barrier()
        base = (sid * 2 + cid) * per_w

        @pl.loop(0, n_ch)
        def _(c):
            off = base + c * _SCH
            pltpu.sync_copy(i_hbm.at[pl.ds(off, _SCH)], idx_v)
            pltpu.sync_copy(m_hbm.at[pl.ds(off, _SCH)], rows_v)
            pltpu.sync_copy(rows_v, acc.at[idx_v], add=True)

        plsc.subcore_barrier()
        pltpu.sync_copy(acc.at[pl.ds(sid * rows_per_sub, rows_per_sub)],
                        o_hbm.at[cid, pl.ds(sid * rows_per_sub, rows_per_sub)])

    return k(m, idx, zrows)


def kernel(node_feats, edge_feats, rigids, node_mask, Wn0, bn0, Wn1, bn1, Wn2,
           bn2, We0, be0, We1, be1, We2, be2, Wma, Wmb, Wmz, Wmr, bm, Wm1, bm1,
           Wnu, bnu, Wea, Web, Wez, beu, Ws, edge_index, batch):
    dst = edge_index[0]
    src = edge_index[1]

    nf = jnp.pad(node_feats, ((0, NP - N), (0, 0)))
    rig = jnp.pad(rigids, ((0, NP - N), (0, 128 - rigids.shape[1])))
    mask = jnp.pad(node_mask, (0, NP - N)).reshape(NP, 1)
    batch3 = jnp.pad(batch, (0, NP - N), constant_values=B).reshape(NP // NBLK, 1, NBLK)
    zrows = jnp.zeros((NP // 16, 128), F32)
    wr_pad = jnp.pad(Wmr, ((0, 0), (0, 1), (0, 0)))

    def r1(b):
        return b.reshape(1, -1)

    rs = _sc_gather(rig, src)
    rd = _sc_gather(rig, dst)
    s = _node_embed(nf, Wn0, r1(bn0), Wn1, r1(bn1), Wn2, r1(bn2))
    z, rel = _edge_embed(edge_feats, rs, rd, We0, r1(be0), We1, r1(be1),
                         We2, r1(be2))

    for l in range(L):
        a, b = _ab(s, Wma[l], Wmb[l])
        asrc = _sc_gather(a, src)
        bdst = _sc_gather(b, dst)
        mlo, mhi = _msg(asrc, bdst, z, rel, Wmz[l], wr_pad[l], r1(bm[l]),
                        Wm1[l], r1(bm1[l]))
        plo = _sc_scatter_add(mlo, dst, zrows)
        phi = _sc_scatter_add(mhi, dst, zrows)
        s, ea, eb = _node_upd(s, plo[0], plo[1], phi[0], phi[1],
                              Wnu[l], r1(bnu[l]), Wea[l], Web[l], mask)
        eas = _sc_gather(ea, src)
        ebd = _sc_gather(eb, dst)
        z = _z_upd(z, eas, ebd, Wez[l], r1(beu[l]))

    _, _, score = _pool(s, mask, batch3, Ws)
    return score
